# baseline (speedup 1.0000x reference)
import functools

import jax
import jax.numpy as jnp
from jax import lax
from jax.experimental import pallas as pl
from jax.experimental.pallas import tpu as pltpu
from jax.experimental.pallas import tpu_sc as plsc

VOCAB = 1000000
DIM = 32
BATCH = 4096
HIST = 50
B = BATCH * HIST

NC = 2
NS = 16
NW = NC * NS
BPW = B // NW
C = 1280
NCHUNK = BPW // C

_mesh = plsc.VectorSubcoreMesh(core_axis_name="c", subcore_axis_name="s")


@functools.partial(
    pl.kernel,
    out_type=jax.ShapeDtypeStruct((B, DIM), jnp.float32),
    mesh=_mesh,
    scratch_types=[
        pltpu.VMEM((BPW,), jnp.int32),
        pltpu.VMEM((2, C, DIM), jnp.float32),
        pltpu.SemaphoreType.DMA,
        pltpu.SemaphoreType.DMA,
    ],
    compiler_params=pltpu.CompilerParams(use_tc_tiling_on_sc=False),
)
def _sc_gather(idx_hbm, table_hbm, out_hbm, idx_v, rows_v, sem0, sem1):
    wid = lax.axis_index("s") * NC + lax.axis_index("c")
    base = wid * BPW
    pltpu.sync_copy(idx_hbm.at[pl.ds(base, BPW)], idx_v)

    sems = (sem0, sem1)

    def issue(j, slot):
        return pltpu.async_copy(
            table_hbm.at[idx_v.at[pl.ds(j * C, C)]], rows_v.at[slot], sems[slot]
        )

    issue(0, 0)
    for j in range(NCHUNK):
        slot = j % 2
        if j + 1 < NCHUNK:
            issue(j + 1, 1 - slot)
        pltpu.make_async_copy(
            table_hbm.at[idx_v.at[pl.ds(j * C, C)]], rows_v.at[slot], sems[slot]
        ).wait()
        pltpu.sync_copy(rows_v.at[slot], out_hbm.at[pl.ds(base + j * C, C)])


def kernel(inputs, table):
    idx_flat = inputs.reshape(B)
    out = _sc_gather(idx_flat, table)
    return out.reshape(BATCH, HIST, DIM)
